# Initial kernel scaffold; baseline (speedup 1.0000x reference)
import jax
import jax.numpy as jnp
from jax.experimental import pallas as pl


def kernel(indices, eye):
    raise NotImplementedError("write your pallas kernel here")


# speedup vs baseline: 1.0072x; 1.0072x over previous
import functools

import jax
import jax.numpy as jnp
from jax import lax
from jax.experimental import pallas as pl
from jax.experimental.pallas import tpu as pltpu
from jax.experimental.pallas import tpu_sc as plsc

try:
    _info = plsc.get_sparse_core_info()
    NUM_CORES, NUM_SUBCORES = int(_info.num_cores), int(_info.num_subcores)
except Exception:
    NUM_CORES, NUM_SUBCORES = 2, 16
NUM_WORKERS = NUM_CORES * NUM_SUBCORES

ZERO_ROWS = 32
IDX_MINOR = 128


@functools.lru_cache(maxsize=None)
def _build(batch, num_classes):
    rows_per_worker = batch // NUM_WORKERS
    assert rows_per_worker * NUM_WORKERS == batch
    assert rows_per_worker % ZERO_ROWS == 0
    assert rows_per_worker % IDX_MINOR == 0
    zero_words = ZERO_ROWS * num_classes
    n_zero_dmas = rows_per_worker // ZERO_ROWS
    n_scatters = rows_per_worker // IDX_MINOR

    mesh = plsc.VectorSubcoreMesh(core_axis_name="c", subcore_axis_name="s")

    @functools.partial(
        pl.kernel,
        mesh=mesh,
        out_type=jax.ShapeDtypeStruct((batch * num_classes,), jnp.float32),
        scratch_types=[
            pltpu.VMEM((zero_words,), jnp.float32),
            pltpu.VMEM((rows_per_worker,), jnp.int32),
            pltpu.VMEM((n_scatters, IDX_MINOR), jnp.int32),
            pltpu.VMEM((IDX_MINOR,), jnp.float32),
            pltpu.SemaphoreType.DMA,
            pltpu.SemaphoreType.DMA,
        ],
    )
    def onehot(idx_hbm, out_hbm, zbuf, idx_v, pos_v, ones_v, sem_z, sem_o):
        wid = lax.axis_index("s") * NUM_CORES + lax.axis_index("c")
        row0 = wid * rows_per_worker

        def zero_body(i, carry):
            for u in range(8):
                zbuf[pl.ds(i * 128 + u * 16, 16)] = jnp.zeros((16,), jnp.float32)
            return carry

        lax.fori_loop(0, zero_words // 128, zero_body, 0)
        for u in range(IDX_MINOR // 16):
            ones_v[pl.ds(u * 16, 16)] = jnp.ones((16,), jnp.float32)

        handles = [
            pltpu.async_copy(
                zbuf,
                out_hbm.at[pl.ds((row0 + c * ZERO_ROWS) * num_classes, zero_words)],
                sem_z,
            )
            for c in range(n_zero_dmas)
        ]

        pltpu.sync_copy(idx_hbm.at[pl.ds(row0, rows_per_worker)], idx_v)
        per_row = IDX_MINOR // 16
        for r in range(rows_per_worker // 16):
            v = idx_v[pl.ds(r * 16, 16)]
            rows = row0 + r * 16 + lax.iota(jnp.int32, 16)
            pos_v[r // per_row, pl.ds((r % per_row) * 16, 16)] = rows * num_classes + v

        for h in handles:
            h.wait()

        for j in range(n_scatters):
            pltpu.async_copy(ones_v, out_hbm.at[pos_v.at[j]], sem_o).wait()

    return onehot


def kernel(indices, eye):
    batch = indices.shape[0]
    num_classes = eye.shape[0]
    out = _build(batch, num_classes)(indices)
    return out.reshape(batch, num_classes)


# speedup vs baseline: 1.5015x; 1.4908x over previous
import functools

import jax
import jax.numpy as jnp
from jax import lax
from jax.experimental import pallas as pl
from jax.experimental.pallas import tpu as pltpu
from jax.experimental.pallas import tpu_sc as plsc

try:
    _info = plsc.get_sparse_core_info()
    NUM_CORES, NUM_SUBCORES = int(_info.num_cores), int(_info.num_subcores)
except Exception:
    NUM_CORES, NUM_SUBCORES = 2, 16
NUM_WORKERS = NUM_CORES * NUM_SUBCORES

CHUNK_ROWS = 32


@functools.lru_cache(maxsize=None)
def _build(batch, num_classes):
    rows_per_worker = batch // NUM_WORKERS
    assert rows_per_worker * NUM_WORKERS == batch
    assert rows_per_worker % CHUNK_ROWS == 0
    n_chunks = rows_per_worker // CHUNK_ROWS

    mesh = plsc.VectorSubcoreMesh(core_axis_name="c", subcore_axis_name="s")

    @functools.partial(
        pl.kernel,
        mesh=mesh,
        out_type=jax.ShapeDtypeStruct((batch, num_classes), jnp.float32),
        scratch_types=[
            pltpu.VMEM((2, CHUNK_ROWS, num_classes), jnp.float32),
            pltpu.VMEM((rows_per_worker,), jnp.int32),
            pltpu.SemaphoreType.DMA,
            pltpu.SemaphoreType.DMA,
        ],
    )
    def onehot(idx_hbm, zeros_hbm, out_hbm, zbuf, idx_v, sem0, sem1):
        wid = lax.axis_index("s") * NUM_CORES + lax.axis_index("c")
        row0 = wid * rows_per_worker
        sems = (sem0, sem1)
        zeros16 = jnp.zeros((16,), jnp.float32)
        iota16 = lax.iota(jnp.int32, 16)

        def put_row(b, row, v, vec):
            c0 = pl.multiple_of((v >> 4) << 4, 16)
            zbuf[b, row, pl.ds(c0, 16)] = vec

        def chunk_rows(c, b, make_vec):
            for g in range(CHUNK_ROWS // 16):
                v16 = idx_v[pl.ds(c * CHUNK_ROWS + g * 16, 16)]
                for j in range(16):
                    v = v16[j]
                    put_row(b, g * 16 + j, v, make_vec(v))

        pltpu.sync_copy(idx_hbm.at[pl.ds(row0, rows_per_worker)], idx_v)
        pltpu.sync_copy(zeros_hbm, zbuf.at[0])
        pltpu.sync_copy(zeros_hbm, zbuf.at[1])

        handles = [None, None]
        for c in range(n_chunks):
            b = c % 2
            if handles[b] is not None:
                handles[b].wait()
                chunk_rows(c - 2, b, lambda v: zeros16)
            chunk_rows(
                c, b,
                lambda v: jnp.where(iota16 == (v & 15), 1.0, 0.0).astype(jnp.float32),
            )
            handles[b] = pltpu.async_copy(
                zbuf.at[b],
                out_hbm.at[pl.ds(row0 + c * CHUNK_ROWS, CHUNK_ROWS)],
                sems[b],
            )
        for b in range(2):
            if handles[b] is not None:
                handles[b].wait()

    return onehot


def kernel(indices, eye):
    batch = indices.shape[0]
    num_classes = eye.shape[0]
    zeros = jnp.zeros((CHUNK_ROWS, num_classes), jnp.float32)
    return _build(batch, num_classes)(indices, zeros)
